# baseline (speedup 1.0000x reference)
import functools

import jax
import jax.numpy as jnp
from jax import lax
from jax.experimental import pallas as pl
from jax.experimental.pallas import tpu as pltpu
from jax.experimental.pallas import tpu_sc as plsc

N_NODES = 10000
D_HID = 128

NC = 2
NS = 16
NW = NC * NS

CH = 128
S = 8
SUP = S * CH
NSUP = 10
E_EDGES = 320000
PER_W = SUP * NSUP
NCHUNK = PER_W // CH
E_PAD = NW * PER_W

N_DEG_PAD = 10240
N_ACC_PAD = 10112


def _deg_body(dst2_hbm, ew_hbm, out_hbm, deg_sp, dstv, ewv, zbuf):
    cid = lax.axis_index("c")
    sid = lax.axis_index("s")
    wid = sid * NC + cid

    def zb(i, _):
        zbuf[pl.ds(i * 16, 16)] = jnp.zeros((16,), jnp.float32)
        return 0

    lax.fori_loop(0, 40, zb, 0)
    pltpu.sync_copy(zbuf, deg_sp.at[pl.ds(sid * 640, 640)])
    plsc.subcore_barrier()

    base0 = wid * PER_W
    row0 = wid * NCHUNK

    def super_chunk(g, _):
        pltpu.sync_copy(ew_hbm.at[pl.ds(base0 + g * SUP, SUP)], ewv)
        pltpu.sync_copy(dst2_hbm.at[pl.ds(row0 + g * S, S)], dstv)
        for c in range(S):
            pltpu.sync_copy(ewv.at[pl.ds(c * CH, CH)],
                            deg_sp.at[dstv.at[c]], add=True)
        return 0

    lax.fori_loop(0, NSUP, super_chunk, 0)
    plsc.subcore_barrier()
    pltpu.sync_copy(deg_sp.at[pl.ds(sid * 640, 640)],
                    out_hbm.at[cid, pl.ds(sid * 640, 640)])


def _spmm_body(hp_hbm, src_hbm, dst2_hbm, ew_hbm, out_hbm,
               acc_sp, srcv, dstv, ewv, rows0, rows1,
               sem0, sem1, ssem0, ssem1):
    cid = lax.axis_index("c")
    sid = lax.axis_index("s")
    wid = sid * NC + cid

    def zb(i, _):
        for j in range(8):
            rows0[i, pl.ds(j * 16, 16)] = jnp.zeros((16,), jnp.float32)
        return 0

    lax.fori_loop(0, CH, zb, 0)
    for k in range(4):
        pltpu.sync_copy(rows0, acc_sp.at[pl.ds(sid * 632 + k * CH, CH)])
    pltpu.sync_copy(rows0.at[pl.ds(0, 120)],
                    acc_sp.at[pl.ds(sid * 632 + 4 * CH, 120)])
    plsc.subcore_barrier()

    base0 = wid * PER_W
    row0 = wid * NCHUNK
    bufs = (rows0, rows1)
    sems = (sem0, sem1)

    def super_chunk(g, _):
        pltpu.sync_copy(src_hbm.at[pl.ds(base0 + g * SUP, SUP)], srcv)
        pltpu.sync_copy(ew_hbm.at[pl.ds(base0 + g * SUP, SUP)], ewv)
        pltpu.sync_copy(dst2_hbm.at[pl.ds(row0 + g * S, S)], dstv)

        ssems = (ssem0, ssem1)
        pending = pltpu.async_copy(
            hp_hbm.at[srcv.at[pl.ds(0, CH)]], rows0, sem0)
        scat = [None, None]
        for c in range(S):
            b = c & 1
            pending.wait()
            if c + 1 < S:
                if scat[1 - b] is not None:
                    scat[1 - b].wait()
                    scat[1 - b] = None
                pending = pltpu.async_copy(
                    hp_hbm.at[srcv.at[pl.ds((c + 1) * CH, CH)]],
                    bufs[1 - b], sems[1 - b])
            off = c * CH
            rb = bufs[b]

            def scale(t, _, off=off, rb=rb):
                ew16 = ewv[pl.ds(off + t * 16, 16)]
                for i in range(16):
                    sc = ew16[i]
                    r = t * 16 + i
                    for j in range(8):
                        sl = pl.ds(j * 16, 16)
                        rb[r, sl] = rb[r, sl] * sc
                return 0

            lax.fori_loop(0, CH // 16, scale, 0)
            scat[b] = pltpu.async_copy(
                rb, acc_sp.at[dstv.at[c]], ssems[b], add=True)
        scat[0].wait()
        scat[1].wait()
        return 0

    lax.fori_loop(0, NSUP, super_chunk, 0)
    plsc.subcore_barrier()
    off = sid * 632
    pltpu.sync_copy(acc_sp.at[pl.ds(off, 632)],
                    out_hbm.at[cid, pl.ds(off, 632)])


_deg_kernel = functools.partial(
    pl.kernel,
    _deg_body,
    out_type=jax.ShapeDtypeStruct((NC, N_DEG_PAD), jnp.float32),
    mesh=plsc.VectorSubcoreMesh(core_axis_name="c", subcore_axis_name="s"),
    scratch_types=[
        pltpu.VMEM_SHARED((N_DEG_PAD,), jnp.float32),
        pltpu.VMEM((S, CH), jnp.int32),
        pltpu.VMEM((SUP,), jnp.float32),
        pltpu.VMEM((640,), jnp.float32),
    ],
)()

_spmm_kernel = functools.partial(
    pl.kernel,
    _spmm_body,
    out_type=jax.ShapeDtypeStruct((NC, N_ACC_PAD, D_HID), jnp.float32),
    mesh=plsc.VectorSubcoreMesh(core_axis_name="c", subcore_axis_name="s"),
    scratch_types=[
        pltpu.VMEM_SHARED((N_ACC_PAD, D_HID), jnp.float32),
        pltpu.VMEM((SUP,), jnp.int32),
        pltpu.VMEM((S, CH), jnp.int32),
        pltpu.VMEM((SUP,), jnp.float32),
        pltpu.VMEM((CH, D_HID), jnp.float32),
        pltpu.VMEM((CH, D_HID), jnp.float32),
        pltpu.SemaphoreType.DMA,
        pltpu.SemaphoreType.DMA,
        pltpu.SemaphoreType.DMA,
        pltpu.SemaphoreType.DMA,
    ],
)()


def _tcb_body(x_ref, w_ref, degp_ref, hp_ref, dinv_ref):
    deg = degp_ref[0] + degp_ref[1] + 1.0
    dinv = jnp.where(deg > 0, lax.rsqrt(deg), 0.0)
    dinv_ref[...] = dinv
    hp_ref[...] = jnp.dot(x_ref[...], w_ref[...],
                          preferred_element_type=jnp.float32) * dinv


def _tcd_body(acc_ref, hp_ref, dinv_ref, w_ref, b_ref, out_ref):
    dinv = dinv_ref[...]
    acc = acc_ref[0, :N_NODES, :] + acc_ref[1, :N_NODES, :]
    pre = dinv * (acc + hp_ref[...]) + b_ref[...]
    h = jnp.maximum(pre, 0.0)
    out_ref[...] = jnp.dot(h, w_ref[...],
                           preferred_element_type=jnp.float32) * dinv


def _tcf_body(acc_ref, hp_ref, dinv_ref, w_ref, b_ref, b3_ref, out_ref):
    dinv = dinv_ref[...]
    acc = acc_ref[0, :N_NODES, :] + acc_ref[1, :N_NODES, :]
    pre = dinv * (acc + hp_ref[...]) + b_ref[...]
    h = jnp.maximum(pre, 0.0)
    out_ref[...] = jnp.dot(h, w_ref[...],
                           preferred_element_type=jnp.float32) + b3_ref[...]


def kernel(x, edge_index, edge_weight, W1, b1, W2, b2, W3, b3):
    src = edge_index[0].astype(jnp.int32)
    dst = edge_index[1].astype(jnp.int32)
    ew = edge_weight.astype(jnp.float32)

    pad = E_PAD - E_EDGES
    zi = jnp.zeros((pad,), jnp.int32)
    src_p = jnp.concatenate([src, zi])
    dst2 = jnp.concatenate([dst, zi]).reshape(E_PAD // CH, CH)
    ew_p = jnp.concatenate([ew, jnp.zeros((pad,), jnp.float32)])

    deg_parts = _deg_kernel(dst2, ew_p)
    degp = deg_parts[:, :N_NODES, None]

    hp1, dinv = pl.pallas_call(
        _tcb_body,
        out_shape=(
            jax.ShapeDtypeStruct((N_NODES, D_HID), jnp.float32),
            jax.ShapeDtypeStruct((N_NODES, 1), jnp.float32),
        ),
    )(x, W1, degp)

    acc1 = _spmm_kernel(hp1, src_p, dst2, ew_p)

    hp2 = pl.pallas_call(
        _tcd_body,
        out_shape=jax.ShapeDtypeStruct((N_NODES, D_HID), jnp.float32),
    )(acc1, hp1, dinv, W2, b1[None, :])

    acc2 = _spmm_kernel(hp2, src_p, dst2, ew_p)

    out = pl.pallas_call(
        _tcf_body,
        out_shape=jax.ShapeDtypeStruct((N_NODES, W3.shape[1]), jnp.float32),
    )(acc2, hp2, dinv, W3, b2[None, :], b3[None, :])

    return out


# speedup vs baseline: 1.2219x; 1.2219x over previous
import functools

import jax
import jax.numpy as jnp
from jax import lax
from jax.experimental import pallas as pl
from jax.experimental.pallas import tpu as pltpu
from jax.experimental.pallas import tpu_sc as plsc

N_NODES = 10000
D_HID = 128

NC = 2
NS = 16
NW = NC * NS

CH = 128
S = 8
GCH = 64
GS = 16
NBUF = 4
SUP = 1024
NSUP = 10
E_EDGES = 320000
PER_W = SUP * NSUP
NCHUNK = PER_W // CH
E_PAD = NW * PER_W

N_DEG_PAD = 10240
N_ACC_PAD = 10112


def _deg_body(dst2_hbm, ew_hbm, out_hbm, deg_sp, dstv, ewv, zbuf):
    cid = lax.axis_index("c")
    sid = lax.axis_index("s")
    wid = sid * NC + cid

    def zb(i, _):
        zbuf[pl.ds(i * 16, 16)] = jnp.zeros((16,), jnp.float32)
        return 0

    lax.fori_loop(0, 40, zb, 0)
    pltpu.sync_copy(zbuf, deg_sp.at[pl.ds(sid * 640, 640)])
    plsc.subcore_barrier()

    base0 = wid * PER_W
    row0 = wid * NCHUNK

    def super_chunk(g, _):
        pltpu.sync_copy(ew_hbm.at[pl.ds(base0 + g * SUP, SUP)], ewv)
        pltpu.sync_copy(dst2_hbm.at[pl.ds(row0 + g * S, S)], dstv)
        for c in range(S):
            pltpu.sync_copy(ewv.at[pl.ds(c * CH, CH)],
                            deg_sp.at[dstv.at[c]], add=True)
        return 0

    lax.fori_loop(0, NSUP, super_chunk, 0)
    plsc.subcore_barrier()
    pltpu.sync_copy(deg_sp.at[pl.ds(sid * 640, 640)],
                    out_hbm.at[cid, pl.ds(sid * 640, 640)])


def _spmm_body(hp_hbm, src_hbm, dst2_hbm, ew_hbm, out_hbm,
               acc_sp, srcv, dstv, ewv,
               rows0, rows1, rows2, rows3,
               sem0, sem1, sem2, sem3, ssem):
    cid = lax.axis_index("c")
    sid = lax.axis_index("s")
    wid = sid * NC + cid
    bufs = (rows0, rows1, rows2, rows3)
    sems = (sem0, sem1, sem2, sem3)

    def zb(i, _):
        for j in range(8):
            rows0[i, pl.ds(j * 16, 16)] = jnp.zeros((16,), jnp.float32)
        return 0

    lax.fori_loop(0, GCH, zb, 0)
    for k in range(9):
        pltpu.sync_copy(rows0, acc_sp.at[pl.ds(sid * 632 + k * GCH, GCH)])
    pltpu.sync_copy(rows0.at[pl.ds(0, 56)],
                    acc_sp.at[pl.ds(sid * 632 + 9 * GCH, 56)])
    plsc.subcore_barrier()

    base0 = wid * PER_W
    row0 = wid * (PER_W // GCH)

    def super_chunk(g, _):
        pltpu.sync_copy(src_hbm.at[pl.ds(base0 + g * SUP, SUP)], srcv)
        pltpu.sync_copy(ew_hbm.at[pl.ds(base0 + g * SUP, SUP)], ewv)
        pltpu.sync_copy(dst2_hbm.at[pl.ds(row0 + g * GS, GS)], dstv)

        grs = [
            pltpu.async_copy(
                hp_hbm.at[srcv.at[pl.ds(k * GCH, GCH)]], bufs[k], sems[k])
            for k in range(NBUF)
        ]
        for c in range(GS):
            b = c % NBUF
            grs[b].wait()
            off = c * GCH
            rb = bufs[b]

            def scale(t, _, off=off, rb=rb):
                ew16 = ewv[pl.ds(off + t * 16, 16)]
                for i in range(16):
                    sc = ew16[i]
                    r = t * 16 + i
                    for j in range(8):
                        sl = pl.ds(j * 16, 16)
                        rb[r, sl] = rb[r, sl] * sc
                return 0

            lax.fori_loop(0, GCH // 16, scale, 0)
            pltpu.async_copy(rb, acc_sp.at[dstv.at[c]], ssem,
                             add=True).wait()
            if c + NBUF < GS:
                grs[b] = pltpu.async_copy(
                    hp_hbm.at[srcv.at[pl.ds((c + NBUF) * GCH, GCH)]],
                    bufs[b], sems[b])
        return 0

    lax.fori_loop(0, NSUP, super_chunk, 0)
    plsc.subcore_barrier()
    off = sid * 632
    pltpu.sync_copy(acc_sp.at[pl.ds(off, 632)],
                    out_hbm.at[cid, pl.ds(off, 632)])


_deg_kernel = functools.partial(
    pl.kernel,
    _deg_body,
    out_type=jax.ShapeDtypeStruct((NC, N_DEG_PAD), jnp.float32),
    mesh=plsc.VectorSubcoreMesh(core_axis_name="c", subcore_axis_name="s"),
    scratch_types=[
        pltpu.VMEM_SHARED((N_DEG_PAD,), jnp.float32),
        pltpu.VMEM((S, CH), jnp.int32),
        pltpu.VMEM((SUP,), jnp.float32),
        pltpu.VMEM((640,), jnp.float32),
    ],
)()

_spmm_kernel = functools.partial(
    pl.kernel,
    _spmm_body,
    out_type=jax.ShapeDtypeStruct((NC, N_ACC_PAD, D_HID), jnp.float32),
    mesh=plsc.VectorSubcoreMesh(core_axis_name="c", subcore_axis_name="s"),
    scratch_types=[
        pltpu.VMEM_SHARED((N_ACC_PAD, D_HID), jnp.float32),
        pltpu.VMEM((SUP,), jnp.int32),
        pltpu.VMEM((GS, GCH), jnp.int32),
        pltpu.VMEM((SUP,), jnp.float32),
        pltpu.VMEM((GCH, D_HID), jnp.float32),
        pltpu.VMEM((GCH, D_HID), jnp.float32),
        pltpu.VMEM((GCH, D_HID), jnp.float32),
        pltpu.VMEM((GCH, D_HID), jnp.float32),
        pltpu.SemaphoreType.DMA,
        pltpu.SemaphoreType.DMA,
        pltpu.SemaphoreType.DMA,
        pltpu.SemaphoreType.DMA,
        pltpu.SemaphoreType.DMA,
    ],
)()


def _tcb_body(x_ref, w_ref, degp_ref, hp_ref, dinv_ref):
    deg = degp_ref[0] + degp_ref[1] + 1.0
    dinv = jnp.where(deg > 0, lax.rsqrt(deg), 0.0)
    dinv_ref[...] = dinv
    hp_ref[...] = jnp.dot(x_ref[...], w_ref[...],
                          preferred_element_type=jnp.float32) * dinv


def _tcd_body(acc_ref, hp_ref, dinv_ref, w_ref, b_ref, out_ref):
    dinv = dinv_ref[...]
    acc = acc_ref[0, :N_NODES, :] + acc_ref[1, :N_NODES, :]
    pre = dinv * (acc + hp_ref[...]) + b_ref[...]
    h = jnp.maximum(pre, 0.0)
    out_ref[...] = jnp.dot(h, w_ref[...],
                           preferred_element_type=jnp.float32) * dinv


def _tcf_body(acc_ref, hp_ref, dinv_ref, w_ref, b_ref, b3_ref, out_ref):
    dinv = dinv_ref[...]
    acc = acc_ref[0, :N_NODES, :] + acc_ref[1, :N_NODES, :]
    pre = dinv * (acc + hp_ref[...]) + b_ref[...]
    h = jnp.maximum(pre, 0.0)
    out_ref[...] = jnp.dot(h, w_ref[...],
                           preferred_element_type=jnp.float32) + b3_ref[...]


def kernel(x, edge_index, edge_weight, W1, b1, W2, b2, W3, b3):
    src = edge_index[0].astype(jnp.int32)
    dst = edge_index[1].astype(jnp.int32)
    ew = edge_weight.astype(jnp.float32)

    pad = E_PAD - E_EDGES
    zi = jnp.zeros((pad,), jnp.int32)
    src_p = jnp.concatenate([src, zi])
    dst_p = jnp.concatenate([dst, zi])
    dst2 = dst_p.reshape(E_PAD // CH, CH)
    dst2g = dst_p.reshape(E_PAD // GCH, GCH)
    ew_p = jnp.concatenate([ew, jnp.zeros((pad,), jnp.float32)])

    deg_parts = _deg_kernel(dst2, ew_p)
    degp = deg_parts[:, :N_NODES, None]

    hp1, dinv = pl.pallas_call(
        _tcb_body,
        out_shape=(
            jax.ShapeDtypeStruct((N_NODES, D_HID), jnp.float32),
            jax.ShapeDtypeStruct((N_NODES, 1), jnp.float32),
        ),
    )(x, W1, degp)

    acc1 = _spmm_kernel(hp1, src_p, dst2g, ew_p)

    hp2 = pl.pallas_call(
        _tcd_body,
        out_shape=jax.ShapeDtypeStruct((N_NODES, D_HID), jnp.float32),
    )(acc1, hp1, dinv, W2, b1[None, :])

    acc2 = _spmm_kernel(hp2, src_p, dst2g, ew_p)

    out = pl.pallas_call(
        _tcf_body,
        out_shape=jax.ShapeDtypeStruct((N_NODES, W3.shape[1]), jnp.float32),
    )(acc2, hp2, dinv, W3, b2[None, :], b3[None, :])

    return out


# speedup vs baseline: 2.3171x; 1.8964x over previous
import functools

import jax
import jax.numpy as jnp
from jax import lax
from jax.experimental import pallas as pl
from jax.experimental.pallas import tpu as pltpu
from jax.experimental.pallas import tpu_sc as plsc

N_NODES = 10000
D_HID = 128

NC = 2
NS = 16
NW = NC * NS

CH = 128
S = 8
GCH = 64
GS = 16
NBUF = 4
SUP = 1024
NSUP = 10
E_EDGES = 320000
PER_W = SUP * NSUP
NCHUNK = PER_W // CH
E_PAD = NW * PER_W
PER_T = E_PAD // NS
NSUPT = PER_T // SUP

N_DEG_PAD = 10240
N_ACC_PAD = 10112


def _deg_body(dst2_hbm, ew_hbm, out_hbm, deg_sp, dstv, ewv, zbuf):
    cid = lax.axis_index("c")
    sid = lax.axis_index("s")
    wid = sid * NC + cid

    def zb(i, _):
        zbuf[pl.ds(i * 16, 16)] = jnp.zeros((16,), jnp.float32)
        return 0

    lax.fori_loop(0, 40, zb, 0)
    pltpu.sync_copy(zbuf, deg_sp.at[pl.ds(sid * 640, 640)])
    plsc.subcore_barrier()

    base0 = wid * PER_W
    row0 = wid * NCHUNK

    def super_chunk(g, _):
        pltpu.sync_copy(ew_hbm.at[pl.ds(base0 + g * SUP, SUP)], ewv)
        pltpu.sync_copy(dst2_hbm.at[pl.ds(row0 + g * S, S)], dstv)
        for c in range(S):
            pltpu.sync_copy(ewv.at[pl.ds(c * CH, CH)],
                            deg_sp.at[dstv.at[c]], add=True)
        return 0

    lax.fori_loop(0, NSUP, super_chunk, 0)
    plsc.subcore_barrier()
    pltpu.sync_copy(deg_sp.at[pl.ds(sid * 640, 640)],
                    out_hbm.at[cid, pl.ds(sid * 640, 640)])


def _spmm_body(hb_hbm, src_hbm, dst2_hbm, ew_hbm, out_hbm,
               acc_sp, hb_sp, srcv, dstv, ewv,
               rows0, rows1, rows2, rows3, stg,
               sem0, sem1, sem2, sem3, ssem):
    cid = lax.axis_index("c")
    sid = lax.axis_index("s")

    pltpu.sync_copy(hb_hbm.at[cid, pl.ds(sid * 632, 632)],
                    hb_sp.at[pl.ds(sid * 632, 632)])

    def zb(i, _):
        for j in range(4):
            stg[i, pl.ds(j * 16, 16)] = jnp.zeros((16,), jnp.float32)
        return 0

    lax.fori_loop(0, GCH, zb, 0)
    for k in range(9):
        pltpu.sync_copy(stg, acc_sp.at[pl.ds(sid * 632 + k * GCH, GCH)])
    pltpu.sync_copy(stg.at[pl.ds(0, 56)],
                    acc_sp.at[pl.ds(sid * 632 + 9 * GCH, 56)])
    plsc.subcore_barrier()

    bufs = (rows0, rows1, rows2, rows3)
    sems = (sem0, sem1, sem2, sem3)
    base0 = sid * PER_T
    row0 = sid * (PER_T // GCH)

    def super_chunk(g, _):
        pltpu.sync_copy(src_hbm.at[pl.ds(base0 + g * SUP, SUP)], srcv)
        pltpu.sync_copy(ew_hbm.at[pl.ds(base0 + g * SUP, SUP)], ewv)
        pltpu.sync_copy(dst2_hbm.at[pl.ds(row0 + g * GS, GS)], dstv)

        grs = [
            pltpu.async_copy(
                hb_sp.at[srcv.at[pl.ds(k * GCH, GCH)]], bufs[k], sems[k])
            for k in range(NBUF)
        ]
        for c in range(GS):
            b = c % NBUF
            grs[b].wait()
            off = c * GCH
            rb = bufs[b]

            def scale(t, _, off=off, rb=rb):
                ew16 = ewv[pl.ds(off + t * 16, 16)]
                for i in range(16):
                    sc = ew16[i]
                    r = t * 16 + i
                    for j in range(2):
                        v = plsc.bitcast(rb[r, pl.ds(j * 16, 16)],
                                         jnp.bfloat16)
                        lo, hi = plsc.unpack(
                            v, format=plsc.PackFormat.INTERLEAVED)
                        stg[r, pl.ds(j * 16, 16)] = lo * sc
                        stg[r, pl.ds(32 + j * 16, 16)] = hi * sc
                return 0

            lax.fori_loop(0, GCH // 16, scale, 0)
            pltpu.async_copy(stg, acc_sp.at[dstv.at[c]], ssem,
                             add=True).wait()
            if c + NBUF < GS:
                grs[b] = pltpu.async_copy(
                    hb_sp.at[srcv.at[pl.ds((c + NBUF) * GCH, GCH)]],
                    bufs[b], sems[b])
        return 0

    lax.fori_loop(0, NSUPT, super_chunk, 0)
    plsc.subcore_barrier()
    off = sid * 632
    pltpu.sync_copy(acc_sp.at[pl.ds(off, 632)],
                    out_hbm.at[cid, pl.ds(off, 632)])


_deg_kernel = functools.partial(
    pl.kernel,
    _deg_body,
    out_type=jax.ShapeDtypeStruct((NC, N_DEG_PAD), jnp.float32),
    mesh=plsc.VectorSubcoreMesh(core_axis_name="c", subcore_axis_name="s"),
    scratch_types=[
        pltpu.VMEM_SHARED((N_DEG_PAD,), jnp.float32),
        pltpu.VMEM((S, CH), jnp.int32),
        pltpu.VMEM((SUP,), jnp.float32),
        pltpu.VMEM((640,), jnp.float32),
    ],
)()

_spmm_kernel = functools.partial(
    pl.kernel,
    _spmm_body,
    out_type=jax.ShapeDtypeStruct((NC, N_ACC_PAD, D_HID // 2), jnp.float32),
    mesh=plsc.VectorSubcoreMesh(core_axis_name="c", subcore_axis_name="s"),
    scratch_types=[
        pltpu.VMEM_SHARED((N_ACC_PAD, D_HID // 2), jnp.float32),
        pltpu.VMEM_SHARED((N_ACC_PAD, D_HID // 4), jnp.int32),
        pltpu.VMEM((SUP,), jnp.int32),
        pltpu.VMEM((GS, GCH), jnp.int32),
        pltpu.VMEM((SUP,), jnp.float32),
        pltpu.VMEM((GCH, D_HID // 4), jnp.int32),
        pltpu.VMEM((GCH, D_HID // 4), jnp.int32),
        pltpu.VMEM((GCH, D_HID // 4), jnp.int32),
        pltpu.VMEM((GCH, D_HID // 4), jnp.int32),
        pltpu.VMEM((GCH, D_HID // 2), jnp.float32),
        pltpu.SemaphoreType.DMA,
        pltpu.SemaphoreType.DMA,
        pltpu.SemaphoreType.DMA,
        pltpu.SemaphoreType.DMA,
        pltpu.SemaphoreType.DMA,
    ],
    compiler_params=pltpu.CompilerParams(needs_layout_passes=False),
)()


def _tcb_body(x_ref, w_ref, degp_ref, hp_ref, dinv_ref):
    deg = degp_ref[0] + degp_ref[1] + 1.0
    dinv = jnp.where(deg > 0, lax.rsqrt(deg), 0.0)
    dinv_ref[...] = dinv
    hp_ref[...] = jnp.dot(x_ref[...], w_ref[...],
                          preferred_element_type=jnp.float32) * dinv


def _tcd_body(acc_ref, hp_ref, dinv_ref, w_ref, b_ref, out_ref):
    dinv = dinv_ref[...]
    acc = jnp.concatenate(
        [acc_ref[0, :N_NODES, :], acc_ref[1, :N_NODES, :]], axis=1)
    pre = dinv * (acc + hp_ref[...]) + b_ref[...]
    h = jnp.maximum(pre, 0.0)
    out_ref[...] = jnp.dot(h, w_ref[...],
                           preferred_element_type=jnp.float32) * dinv


def _tcf_body(acc_ref, hp_ref, dinv_ref, w_ref, b_ref, b3_ref, out_ref):
    dinv = dinv_ref[...]
    acc = jnp.concatenate(
        [acc_ref[0, :N_NODES, :], acc_ref[1, :N_NODES, :]], axis=1)
    pre = dinv * (acc + hp_ref[...]) + b_ref[...]
    h = jnp.maximum(pre, 0.0)
    out_ref[...] = jnp.dot(h, w_ref[...],
                           preferred_element_type=jnp.float32) + b3_ref[...]


def _pack_halves(hp):
    u = lax.bitcast_convert_type(
        hp.astype(jnp.bfloat16), jnp.uint16).astype(jnp.uint32)
    q = D_HID // 4
    halves = jnp.stack([
        u[:, 0:q] | (u[:, q:2 * q] << 16),
        u[:, 2 * q:3 * q] | (u[:, 3 * q:] << 16),
    ]).astype(jnp.int32)
    return jnp.pad(halves, ((0, 0), (0, N_ACC_PAD - N_NODES), (0, 0)))


def kernel(x, edge_index, edge_weight, W1, b1, W2, b2, W3, b3):
    src = edge_index[0].astype(jnp.int32)
    dst = edge_index[1].astype(jnp.int32)
    ew = edge_weight.astype(jnp.float32)

    pad = E_PAD - E_EDGES
    zi = jnp.zeros((pad,), jnp.int32)
    src_p = jnp.concatenate([src, zi])
    dst_p = jnp.concatenate([dst, zi])
    dst2 = dst_p.reshape(E_PAD // CH, CH)
    dst2g = dst_p.reshape(E_PAD // GCH, GCH)
    ew_p = jnp.concatenate([ew, jnp.zeros((pad,), jnp.float32)])

    deg_parts = _deg_kernel(dst2, ew_p)
    degp = deg_parts[:, :N_NODES, None]

    hp1, dinv = pl.pallas_call(
        _tcb_body,
        out_shape=(
            jax.ShapeDtypeStruct((N_NODES, D_HID), jnp.float32),
            jax.ShapeDtypeStruct((N_NODES, 1), jnp.float32),
        ),
    )(x, W1, degp)

    acc1 = _spmm_kernel(_pack_halves(hp1), src_p, dst2g, ew_p)

    hp2 = pl.pallas_call(
        _tcd_body,
        out_shape=jax.ShapeDtypeStruct((N_NODES, D_HID), jnp.float32),
    )(acc1, hp1, dinv, W2, b1[None, :])

    acc2 = _spmm_kernel(_pack_halves(hp2), src_p, dst2g, ew_p)

    out = pl.pallas_call(
        _tcf_body,
        out_shape=jax.ShapeDtypeStruct((N_NODES, W3.shape[1]), jnp.float32),
    )(acc2, hp2, dinv, W3, b2[None, :], b3[None, :])

    return out
